# baseline (speedup 1.0000x reference)
import functools

import jax
import jax.numpy as jnp
from jax import lax
from jax.experimental import pallas as pl
from jax.experimental.pallas import tpu as pltpu
from jax.experimental.pallas import tpu_sc as plsc

B = 2048
N0 = 10
N1 = 10
D = 256


NW = 32
OUT_PER_W = (B * N0) // NW
CH = 8
NB = 2
NCH = OUT_PER_W // CH
LANES = 16


def _sc_mean_body(h2_hbm, m2_hbm, *scratch):
    bufs = scratch[:NB]
    outs = scratch[NB:2 * NB]
    rsems = scratch[2 * NB:3 * NB]
    wsems = scratch[3 * NB:4 * NB]

    wid = lax.axis_index("s") * 2 + lax.axis_index("c")
    in_base = wid * (OUT_PER_W * N1)
    out_base = wid * OUT_PER_W

    for pb in range(NB):
        pltpu.async_copy(
            h2_hbm.at[pl.ds(in_base + pb * CH * N1, CH * N1)], bufs[pb],
            rsems[pb])

    def ring_body(gp, carry):
        for pb in range(NB):
            buf, outb, rsem, wsem = bufs[pb], outs[pb], rsems[pb], wsems[pb]
            g = NB * gp + pb
            pltpu.make_async_copy(
                h2_hbm.at[pl.ds(in_base, CH * N1)], buf, rsem).wait()
            @pl.when(g >= NB)
            def _():
                pltpu.make_async_copy(
                    outb, m2_hbm.at[pl.ds(out_base, CH)], wsem).wait()

            for r in range(CH):
                base = r * N1
                for cb in range(D // LANES):
                    sl = pl.ds(cb * LANES, LANES)
                    t0 = buf[base + 0, sl] + buf[base + 1, sl]
                    t1 = buf[base + 2, sl] + buf[base + 3, sl]
                    t2 = buf[base + 4, sl] + buf[base + 5, sl]
                    t3 = buf[base + 6, sl] + buf[base + 7, sl]
                    t4 = buf[base + 8, sl] + buf[base + 9, sl]
                    outb[r, sl] = ((t0 + t1) + (t2 + t3)) + t4

            pltpu.async_copy(
                outb, m2_hbm.at[pl.ds(out_base + g * CH, CH)], wsem)

            @pl.when(g + NB < NCH)
            def _():
                pltpu.async_copy(
                    h2_hbm.at[pl.ds(in_base + (g + NB) * CH * N1, CH * N1)],
                    buf, rsem)
        return carry

    lax.fori_loop(0, NCH // NB, ring_body, 0)

    for pb in range(NB):
        pltpu.make_async_copy(
            outs[pb], m2_hbm.at[pl.ds(out_base, CH)], wsems[pb]).wait()


def _sc_segment_mean(h2):
    mesh = plsc.VectorSubcoreMesh(core_axis_name="c", subcore_axis_name="s")
    f = functools.partial(
        pl.kernel,
        mesh=mesh,
        out_type=jax.ShapeDtypeStruct((B * N0, D), jnp.float32),
        scratch_types=(
            [pltpu.VMEM((CH * N1, D), jnp.float32)] * NB
            + [pltpu.VMEM((CH, D), jnp.float32)] * NB
            + [pltpu.SemaphoreType.DMA] * (2 * NB)
        ),
    )(_sc_mean_body)
    return f(h2)


R = 128


def _tc_sage_kernel(h0_ref, h1f_ref, m2_ref, w0_ref, w1_ref, m_ref, out_ref):
    x1 = jnp.concatenate([h1f_ref[...], m2_ref[...]], axis=1)
    out1 = jnp.maximum(
        jnp.dot(x1, w0_ref[...], preferred_element_type=jnp.float32), 0.0)

    m1 = jnp.dot(m_ref[...], h1f_ref[...], preferred_element_type=jnp.float32)

    x0 = jnp.concatenate([h0_ref[...], m1], axis=1)
    out0 = jnp.maximum(
        jnp.dot(x0, w0_ref[...], preferred_element_type=jnp.float32), 0.0)

    mo1 = jnp.dot(m_ref[...], out1, preferred_element_type=jnp.float32)

    y = jnp.concatenate([out0, mo1], axis=1)
    out_ref[...] = jnp.dot(y, w1_ref[...], preferred_element_type=jnp.float32)


@jax.jit
def kernel(h0, h1, h2, W_self_0, W_neigh_0, W_self_1, W_neigh_1):
    m2 = _sc_segment_mean(h2)

    w0 = jnp.concatenate([W_self_0, W_neigh_0 * (1.0 / N1)], axis=0)
    w1 = jnp.concatenate([W_self_1, W_neigh_1 * (1.0 / N0)], axis=0)
    m = jnp.repeat(jnp.eye(R, dtype=jnp.float32), N0, axis=1)

    grid = (B // R,)
    return pl.pallas_call(
        _tc_sage_kernel,
        grid=grid,
        in_specs=[
            pl.BlockSpec((R, D), lambda i: (i, 0)),
            pl.BlockSpec((R * N0, D), lambda i: (i, 0)),
            pl.BlockSpec((R * N0, D), lambda i: (i, 0)),
            pl.BlockSpec((2 * D, D), lambda i: (0, 0)),
            pl.BlockSpec((2 * D, D), lambda i: (0, 0)),
            pl.BlockSpec((R, R * N0), lambda i: (0, 0)),
        ],
        out_specs=pl.BlockSpec((R, D), lambda i: (i, 0)),
        out_shape=jax.ShapeDtypeStruct((B, D), jnp.float32),
        compiler_params=pltpu.CompilerParams(
            dimension_semantics=("arbitrary",)),
    )(h0, h1, m2, w0, w1, m)


# speedup vs baseline: 2.1748x; 2.1748x over previous
import functools

import jax
import jax.numpy as jnp
from jax import lax
from jax.experimental import pallas as pl
from jax.experimental.pallas import tpu as pltpu
from jax.experimental.pallas import tpu_sc as plsc

B = 2048
N0 = 10
N1 = 10
D = 256


NW = 32
OUT_PER_W = (B * N0) // NW
CH = 16
NB = 2
NCH = OUT_PER_W // CH
LANES = 16


def _sc_mean_body(h2_hbm, m2_hbm, *scratch):
    bufs = scratch[:NB]
    outs = scratch[NB:2 * NB]
    rsems = scratch[2 * NB:3 * NB]
    wsems = scratch[3 * NB:4 * NB]

    wid = lax.axis_index("s") * 2 + lax.axis_index("c")
    in_base = wid * (OUT_PER_W * N1)
    out_base = wid * OUT_PER_W

    for pb in range(NB):
        pltpu.async_copy(
            h2_hbm.at[pl.ds(in_base + pb * CH * N1, CH * N1)], bufs[pb],
            rsems[pb])

    def ring_body(gp, carry):
        for pb in range(NB):
            buf, outb, rsem, wsem = bufs[pb], outs[pb], rsems[pb], wsems[pb]
            g = NB * gp + pb
            pltpu.make_async_copy(
                h2_hbm.at[pl.ds(in_base, CH * N1)], buf, rsem).wait()
            @pl.when(g >= NB)
            def _():
                pltpu.make_async_copy(
                    outb, m2_hbm.at[pl.ds(out_base, CH)], wsem).wait()

            def row_body(r, c):
                base = r * N1
                for cb in range(D // LANES):
                    sl = pl.ds(cb * LANES, LANES)
                    t0 = buf[base + 0, sl] + buf[base + 1, sl]
                    t1 = buf[base + 2, sl] + buf[base + 3, sl]
                    t2 = buf[base + 4, sl] + buf[base + 5, sl]
                    t3 = buf[base + 6, sl] + buf[base + 7, sl]
                    t4 = buf[base + 8, sl] + buf[base + 9, sl]
                    outb[r, sl] = ((t0 + t1) + (t2 + t3)) + t4
                return c

            lax.fori_loop(0, CH, row_body, 0)

            pltpu.async_copy(
                outb, m2_hbm.at[pl.ds(out_base + g * CH, CH)], wsem)

            @pl.when(g + NB < NCH)
            def _():
                pltpu.async_copy(
                    h2_hbm.at[pl.ds(in_base + (g + NB) * CH * N1, CH * N1)],
                    buf, rsem)
        return carry

    lax.fori_loop(0, NCH // NB, ring_body, 0)

    for pb in range(NB):
        pltpu.make_async_copy(
            outs[pb], m2_hbm.at[pl.ds(out_base, CH)], wsems[pb]).wait()


def _sc_segment_mean(h2):
    mesh = plsc.VectorSubcoreMesh(core_axis_name="c", subcore_axis_name="s")
    f = functools.partial(
        pl.kernel,
        mesh=mesh,
        out_type=jax.ShapeDtypeStruct((B * N0, D), jnp.float32),
        scratch_types=(
            [pltpu.VMEM((CH * N1, D), jnp.float32)] * NB
            + [pltpu.VMEM((CH, D), jnp.float32)] * NB
            + [pltpu.SemaphoreType.DMA] * (2 * NB)
        ),
    )(_sc_mean_body)
    return f(h2)


R = 128


def _tc_sage_kernel(h0_ref, h1f_ref, m2_ref, w0_ref, w1_ref, m_ref, out_ref):
    x1 = jnp.concatenate([h1f_ref[...], m2_ref[...]], axis=1)
    out1 = jnp.maximum(
        jnp.dot(x1, w0_ref[...], preferred_element_type=jnp.float32), 0.0)

    m1 = jnp.dot(m_ref[...], h1f_ref[...], preferred_element_type=jnp.float32)

    x0 = jnp.concatenate([h0_ref[...], m1], axis=1)
    out0 = jnp.maximum(
        jnp.dot(x0, w0_ref[...], preferred_element_type=jnp.float32), 0.0)

    mo1 = jnp.dot(m_ref[...], out1, preferred_element_type=jnp.float32)

    y = jnp.concatenate([out0, mo1], axis=1)
    out_ref[...] = jnp.dot(y, w1_ref[...], preferred_element_type=jnp.float32)


@jax.jit
def kernel(h0, h1, h2, W_self_0, W_neigh_0, W_self_1, W_neigh_1):
    m2 = _sc_segment_mean(h2)

    w0 = jnp.concatenate([W_self_0, W_neigh_0 * (1.0 / N1)], axis=0)
    w1 = jnp.concatenate([W_self_1, W_neigh_1 * (1.0 / N0)], axis=0)
    m = jnp.repeat(jnp.eye(R, dtype=jnp.float32), N0, axis=1)

    grid = (B // R,)
    return pl.pallas_call(
        _tc_sage_kernel,
        grid=grid,
        in_specs=[
            pl.BlockSpec((R, D), lambda i: (i, 0)),
            pl.BlockSpec((R * N0, D), lambda i: (i, 0)),
            pl.BlockSpec((R * N0, D), lambda i: (i, 0)),
            pl.BlockSpec((2 * D, D), lambda i: (0, 0)),
            pl.BlockSpec((2 * D, D), lambda i: (0, 0)),
            pl.BlockSpec((R, R * N0), lambda i: (0, 0)),
        ],
        out_specs=pl.BlockSpec((R, D), lambda i: (i, 0)),
        out_shape=jax.ShapeDtypeStruct((B, D), jnp.float32),
        compiler_params=pltpu.CompilerParams(
            dimension_semantics=("arbitrary",)),
    )(h0, h1, m2, w0, w1, m)


# speedup vs baseline: 2.3886x; 1.0983x over previous
import functools

import jax
import jax.numpy as jnp
from jax import lax
from jax.experimental import pallas as pl
from jax.experimental.pallas import tpu as pltpu
from jax.experimental.pallas import tpu_sc as plsc

B = 2048
N0 = 10
N1 = 10
D = 256


NW = 32
OUT_PER_W = (B * N0) // NW
CH = 16
NB = 2
NCH = OUT_PER_W // CH
LANES = 16


def _sc_mean_body(h2_hbm, m2_hbm, *scratch):
    bufs = scratch[:NB]
    outs = scratch[NB:2 * NB]
    rsems = scratch[2 * NB:3 * NB]
    wsems = scratch[3 * NB:4 * NB]

    wid = lax.axis_index("s") * 2 + lax.axis_index("c")
    in_base = wid * (OUT_PER_W * N1)
    out_base = wid * OUT_PER_W

    for pb in range(NB):
        pltpu.async_copy(
            h2_hbm.at[pl.ds(in_base + pb * CH * N1, CH * N1)], bufs[pb],
            rsems[pb])

    def ring_body(gp, carry):
        for pb in range(NB):
            buf, outb, rsem, wsem = bufs[pb], outs[pb], rsems[pb], wsems[pb]
            g = NB * gp + pb
            pltpu.make_async_copy(
                h2_hbm.at[pl.ds(in_base, CH * N1)], buf, rsem).wait()
            @pl.when(g >= NB)
            def _():
                pltpu.make_async_copy(
                    outb, m2_hbm.at[pl.ds(out_base, CH)], wsem).wait()

            @plsc.parallel_loop(0, CH, 1, unroll=2)
            def _(r):
                base = r * N1
                for cb in range(D // LANES):
                    sl = pl.ds(cb * LANES, LANES)
                    t0 = buf[base + 0, sl] + buf[base + 1, sl]
                    t1 = buf[base + 2, sl] + buf[base + 3, sl]
                    t2 = buf[base + 4, sl] + buf[base + 5, sl]
                    t3 = buf[base + 6, sl] + buf[base + 7, sl]
                    t4 = buf[base + 8, sl] + buf[base + 9, sl]
                    outb[r, sl] = ((t0 + t1) + (t2 + t3)) + t4

            pltpu.async_copy(
                outb, m2_hbm.at[pl.ds(out_base + g * CH, CH)], wsem)

            @pl.when(g + NB < NCH)
            def _():
                pltpu.async_copy(
                    h2_hbm.at[pl.ds(in_base + (g + NB) * CH * N1, CH * N1)],
                    buf, rsem)
        return carry

    lax.fori_loop(0, NCH // NB, ring_body, 0)

    for pb in range(NB):
        pltpu.make_async_copy(
            outs[pb], m2_hbm.at[pl.ds(out_base, CH)], wsems[pb]).wait()


def _sc_segment_mean(h2):
    mesh = plsc.VectorSubcoreMesh(core_axis_name="c", subcore_axis_name="s")
    f = functools.partial(
        pl.kernel,
        mesh=mesh,
        out_type=jax.ShapeDtypeStruct((B * N0, D), jnp.float32),
        scratch_types=(
            [pltpu.VMEM((CH * N1, D), jnp.float32)] * NB
            + [pltpu.VMEM((CH, D), jnp.float32)] * NB
            + [pltpu.SemaphoreType.DMA] * (2 * NB)
        ),
    )(_sc_mean_body)
    return f(h2)


R = 128


def _tc_sage_kernel(h0_ref, h1f_ref, m2_ref, w0_ref, w1_ref, m_ref, out_ref):
    x1 = jnp.concatenate([h1f_ref[...], m2_ref[...]], axis=1)
    out1 = jnp.maximum(
        jnp.dot(x1, w0_ref[...], preferred_element_type=jnp.float32), 0.0)

    m1 = jnp.dot(m_ref[...], h1f_ref[...], preferred_element_type=jnp.float32)

    x0 = jnp.concatenate([h0_ref[...], m1], axis=1)
    out0 = jnp.maximum(
        jnp.dot(x0, w0_ref[...], preferred_element_type=jnp.float32), 0.0)

    mo1 = jnp.dot(m_ref[...], out1, preferred_element_type=jnp.float32)

    y = jnp.concatenate([out0, mo1], axis=1)
    out_ref[...] = jnp.dot(y, w1_ref[...], preferred_element_type=jnp.float32)


@jax.jit
def kernel(h0, h1, h2, W_self_0, W_neigh_0, W_self_1, W_neigh_1):
    m2 = _sc_segment_mean(h2)

    w0 = jnp.concatenate([W_self_0, W_neigh_0 * (1.0 / N1)], axis=0)
    w1 = jnp.concatenate([W_self_1, W_neigh_1 * (1.0 / N0)], axis=0)
    m = jnp.repeat(jnp.eye(R, dtype=jnp.float32), N0, axis=1)

    grid = (B // R,)
    return pl.pallas_call(
        _tc_sage_kernel,
        grid=grid,
        in_specs=[
            pl.BlockSpec((R, D), lambda i: (i, 0)),
            pl.BlockSpec((R * N0, D), lambda i: (i, 0)),
            pl.BlockSpec((R * N0, D), lambda i: (i, 0)),
            pl.BlockSpec((2 * D, D), lambda i: (0, 0)),
            pl.BlockSpec((2 * D, D), lambda i: (0, 0)),
            pl.BlockSpec((R, R * N0), lambda i: (0, 0)),
        ],
        out_specs=pl.BlockSpec((R, D), lambda i: (i, 0)),
        out_shape=jax.ShapeDtypeStruct((B, D), jnp.float32),
        compiler_params=pltpu.CompilerParams(
            dimension_semantics=("arbitrary",)),
    )(h0, h1, m2, w0, w1, m)


# speedup vs baseline: 2.4846x; 1.0402x over previous
import functools

import jax
import jax.numpy as jnp
from jax import lax
from jax.experimental import pallas as pl
from jax.experimental.pallas import tpu as pltpu
from jax.experimental.pallas import tpu_sc as plsc

B = 2048
N0 = 10
N1 = 10
D = 256


NW = 32
OUT_PER_W = (B * N0) // NW
CH = 16
NB = 2
NCH = OUT_PER_W // CH
LANES = 16


def _sc_mean_body(h2_hbm, m2_hbm, *scratch):
    bufs = scratch[:NB]
    outs = scratch[NB:2 * NB]
    rsems = scratch[2 * NB:3 * NB]
    wsems = scratch[3 * NB:4 * NB]

    wid = lax.axis_index("s") * 2 + lax.axis_index("c")
    in_base = wid * (OUT_PER_W * N1)
    out_base = wid * OUT_PER_W

    for pb in range(NB):
        pltpu.async_copy(
            h2_hbm.at[pl.ds(in_base + pb * CH * N1, CH * N1)], bufs[pb],
            rsems[pb])

    def ring_body(gp, carry):
        for pb in range(NB):
            buf, outb, rsem, wsem = bufs[pb], outs[pb], rsems[pb], wsems[pb]
            g = NB * gp + pb
            pltpu.make_async_copy(
                h2_hbm.at[pl.ds(in_base, CH * N1)], buf, rsem).wait()
            @pl.when(g >= NB)
            def _():
                pltpu.make_async_copy(
                    outb, m2_hbm.at[pl.ds(out_base, CH)], wsem).wait()

            @plsc.parallel_loop(0, CH, 1, unroll=4)
            def _(r):
                base = r * N1
                for cb in range(D // LANES):
                    sl = pl.ds(cb * LANES, LANES)
                    t0 = buf[base + 0, sl] + buf[base + 1, sl]
                    t1 = buf[base + 2, sl] + buf[base + 3, sl]
                    t2 = buf[base + 4, sl] + buf[base + 5, sl]
                    t3 = buf[base + 6, sl] + buf[base + 7, sl]
                    t4 = buf[base + 8, sl] + buf[base + 9, sl]
                    outb[r, sl] = ((t0 + t1) + (t2 + t3)) + t4

            pltpu.async_copy(
                outb, m2_hbm.at[pl.ds(out_base + g * CH, CH)], wsem)

            @pl.when(g + NB < NCH)
            def _():
                pltpu.async_copy(
                    h2_hbm.at[pl.ds(in_base + (g + NB) * CH * N1, CH * N1)],
                    buf, rsem)
        return carry

    lax.fori_loop(0, NCH // NB, ring_body, 0)

    for pb in range(NB):
        pltpu.make_async_copy(
            outs[pb], m2_hbm.at[pl.ds(out_base, CH)], wsems[pb]).wait()


def _sc_segment_mean(h2):
    mesh = plsc.VectorSubcoreMesh(core_axis_name="c", subcore_axis_name="s")
    f = functools.partial(
        pl.kernel,
        mesh=mesh,
        out_type=jax.ShapeDtypeStruct((B * N0, D), jnp.float32),
        scratch_types=(
            [pltpu.VMEM((CH * N1, D), jnp.float32)] * NB
            + [pltpu.VMEM((CH, D), jnp.float32)] * NB
            + [pltpu.SemaphoreType.DMA] * (2 * NB)
        ),
    )(_sc_mean_body)
    return f(h2)


R = 128


def _tc_sage_kernel(h0_ref, h1f_ref, m2_ref, w0_ref, w1_ref, m_ref, out_ref):
    x1 = jnp.concatenate([h1f_ref[...], m2_ref[...]], axis=1)
    out1 = jnp.maximum(
        jnp.dot(x1, w0_ref[...], preferred_element_type=jnp.float32), 0.0)

    m1 = jnp.dot(m_ref[...], h1f_ref[...], preferred_element_type=jnp.float32)

    x0 = jnp.concatenate([h0_ref[...], m1], axis=1)
    out0 = jnp.maximum(
        jnp.dot(x0, w0_ref[...], preferred_element_type=jnp.float32), 0.0)

    mo1 = jnp.dot(m_ref[...], out1, preferred_element_type=jnp.float32)

    y = jnp.concatenate([out0, mo1], axis=1)
    out_ref[...] = jnp.dot(y, w1_ref[...], preferred_element_type=jnp.float32)


@jax.jit
def kernel(h0, h1, h2, W_self_0, W_neigh_0, W_self_1, W_neigh_1):
    m2 = _sc_segment_mean(h2)

    w0 = jnp.concatenate([W_self_0, W_neigh_0 * (1.0 / N1)], axis=0)
    w1 = jnp.concatenate([W_self_1, W_neigh_1 * (1.0 / N0)], axis=0)
    m = jnp.repeat(jnp.eye(R, dtype=jnp.float32), N0, axis=1)

    grid = (B // R,)
    return pl.pallas_call(
        _tc_sage_kernel,
        grid=grid,
        in_specs=[
            pl.BlockSpec((R, D), lambda i: (i, 0)),
            pl.BlockSpec((R * N0, D), lambda i: (i, 0)),
            pl.BlockSpec((R * N0, D), lambda i: (i, 0)),
            pl.BlockSpec((2 * D, D), lambda i: (0, 0)),
            pl.BlockSpec((2 * D, D), lambda i: (0, 0)),
            pl.BlockSpec((R, R * N0), lambda i: (0, 0)),
        ],
        out_specs=pl.BlockSpec((R, D), lambda i: (i, 0)),
        out_shape=jax.ShapeDtypeStruct((B, D), jnp.float32),
        compiler_params=pltpu.CompilerParams(
            dimension_semantics=("arbitrary",)),
    )(h0, h1, m2, w0, w1, m)
